# Initial kernel scaffold; baseline (speedup 1.0000x reference)
import jax
import jax.numpy as jnp
from jax.experimental import pallas as pl


def kernel(sc1, sh1, sc2, sh2, w1, w2, wln, bln, wc1_aux, wc1_y, bc1, wc2, bc2, wa1_aux, wa1_y, ba1, wa2, ba2, x0, x1, x2):
    raise NotImplementedError("write your pallas kernel here")


# speedup vs baseline: 6.6495x; 6.6495x over previous
import jax
import jax.numpy as jnp
from jax.experimental import pallas as pl
from jax.experimental.pallas import tpu as pltpu

BB = 128
AUXD = 8 + 3 * 4


def _fused_kernel(x_ref, aux_ref,
                  w1_ref, sc1_ref, sh1_ref, w2_ref, sc2_ref, sh2_ref,
                  wln_ref, bln_ref,
                  wc1a_ref, wc1y_ref, bc1_ref, wc2_ref, bc2_ref,
                  wa1a_ref, wa1y_ref, ba1_ref, wa2_ref, ba2_ref,
                  value_ref, policy_ref,
                  xp_ref, a1p_ref, r1_ref, r2_ref):
    f32 = jnp.float32
    bb = x_ref.shape[-1]

    zc = jnp.zeros((66, 1, bb), f32)
    xp_ref[:, 0:1, :] = zc
    xp_ref[:, 65:66, :] = zc
    zr = jnp.zeros((1, 66, bb), f32)
    xp_ref[0:1, :, :] = zr
    xp_ref[65:66, :, :] = zr
    xp_ref[1:65, 1:65, :] = x_ref[...]

    a1p_ref[...] = jnp.zeros_like(a1p_ref)
    for co in range(4):
        acc = None
        for dh in range(3):
            for dw in range(3):
                w = w1_ref[(co * 3 + dh) * 3 + dw] * sc1_ref[co]
                term = w * xp_ref[dh:dh + 64, dw:dw + 64, :]
                acc = term if acc is None else acc + term
        y = jnp.maximum(acc + sh1_ref[co], 0.0)
        r1_ref[...] = jnp.max(y.reshape(32, 2, 64, bb), axis=1)
        a1p_ref[co, 1:33, 1:33, :] = jnp.maximum(
            r1_ref[:, 0::2, :], r1_ref[:, 1::2, :])

    hparts = []
    for co in range(4):
        acc = None
        for ci in range(4):
            for dh in range(3):
                for dw in range(3):
                    w = w2_ref[((co * 4 + ci) * 3 + dh) * 3 + dw] * sc2_ref[co]
                    term = w * a1p_ref[ci, dh:dh + 32, dw:dw + 32, :]
                    acc = term if acc is None else acc + term
        y = jnp.maximum(acc + sh2_ref[co], 0.0)
        r2_ref[...] = jnp.max(y.reshape(16, 2, 32, bb), axis=1)
        p = jnp.maximum(r2_ref[:, 0::2, :], r2_ref[:, 1::2, :])
        hparts.append(p.reshape(256, bb))
    h = jnp.concatenate(hparts, axis=0)

    y32 = jnp.dot(wln_ref[...], h, preferred_element_type=f32) + bln_ref[...]
    aux = aux_ref[...]
    hc = jnp.maximum(
        jnp.dot(wc1a_ref[...], aux, preferred_element_type=f32)
        + jnp.dot(wc1y_ref[...], y32, preferred_element_type=f32)
        + bc1_ref[...], 0.0)
    value_ref[...] = (jnp.dot(wc2_ref[...], hc, preferred_element_type=f32)
                      + bc2_ref[...])
    ha = jnp.maximum(
        jnp.dot(wa1a_ref[...], aux, preferred_element_type=f32)
        + jnp.dot(wa1y_ref[...], y32, preferred_element_type=f32)
        + ba1_ref[...], 0.0)
    logits = (jnp.dot(wa2_ref[...], ha, preferred_element_type=f32)
              + ba2_ref[...])
    m = jnp.max(logits, axis=0, keepdims=True)
    e = jnp.exp(logits - m)
    policy_ref[...] = e / jnp.sum(e, axis=0, keepdims=True)


def kernel(sc1, sh1, sc2, sh2, w1, w2, wln, bln, wc1_aux, wc1_y, bc1, wc2, bc2,
           wa1_aux, wa1_y, ba1, wa2, ba2, x0, x1, x2):
    b = x2.shape[0]
    na = wa2.shape[1]
    xt = x2.reshape(b, 64 * 64).T.reshape(64, 64, b)
    aux_t = jnp.concatenate([x0.reshape(b, -1), x1.reshape(b, -1)], axis=1).T

    smem = pl.BlockSpec(memory_space=pltpu.MemorySpace.SMEM)
    vmem = pl.BlockSpec(memory_space=pltpu.MemorySpace.VMEM)
    value_t, policy_t = pl.pallas_call(
        _fused_kernel,
        out_shape=(jax.ShapeDtypeStruct((1, b), jnp.float32),
                   jax.ShapeDtypeStruct((na, b), jnp.float32)),
        grid=(b // BB,),
        in_specs=[
            pl.BlockSpec((64, 64, BB), lambda i: (0, 0, i)),
            pl.BlockSpec((AUXD, BB), lambda i: (0, i)),
            smem, smem, smem, smem, smem, smem,
            vmem, vmem, vmem, vmem, vmem, vmem, vmem,
            vmem, vmem, vmem, vmem, vmem,
        ],
        out_specs=(pl.BlockSpec((1, BB), lambda i: (0, i)),
                   pl.BlockSpec((na, BB), lambda i: (0, i))),
        scratch_shapes=[pltpu.VMEM((66, 66, BB), jnp.float32),
                        pltpu.VMEM((4, 34, 34, BB), jnp.float32),
                        pltpu.VMEM((32, 64, BB), jnp.float32),
                        pltpu.VMEM((16, 32, BB), jnp.float32)],
        compiler_params=pltpu.CompilerParams(dimension_semantics=("parallel",)),
    )(xt, aux_t, w1, sc1, sh1, w2, sc2, sh2,
      wln.T, bln.reshape(-1, 1),
      wc1_aux.T, wc1_y.T, bc1.reshape(-1, 1), wc2, bc2,
      wa1_aux.T, wa1_y.T, ba1.reshape(-1, 1), wa2.T, ba2.reshape(-1, 1))
    return value_t.T, policy_t.T
